# baseline (speedup 1.0000x reference)
import functools

import jax
import jax.numpy as jnp
from jax import lax
from jax.experimental import pallas as pl
from jax.experimental.pallas import tpu as pltpu
from jax.experimental.pallas import tpu_sc as plsc

_NUM_ACT = 192
_SIZE = 256
_RANGE = 4.0
_GRID = 2.0 * _RANGE / (_SIZE - 1)
_INV_GRID = 1.0 / _GRID
_HALF_GRID = _GRID / 2.0

_H = 224
_SLAB = _H * _H
_N = 2 * _NUM_ACT * _SLAB
_TBL = _NUM_ACT * _SIZE

_NC, _NS = 2, 16
_NW = _NC * _NS
_CHUNK = _SLAB // 4
_CHUNKS_PER_SLAB = _SLAB // _CHUNK
_TOTAL_CHUNKS = _N // _CHUNK
_CHUNKS_PER_W = _TOTAL_CHUNKS // _NW


def _prep_body(cs_ref, tbl_ref):
    cs = cs_ref[...]
    cs_next = jnp.concatenate([cs[:, 1:], cs[:, _SIZE - 1:]], axis=1)
    slopes = jnp.maximum(cs_next - cs, 0.0)
    col = lax.broadcasted_iota(jnp.int32, (_NUM_ACT, _SIZE), 1)
    slopes = jnp.where((col == 0) | (col >= _SIZE - 2), 0.0, slopes)
    r = lax.broadcasted_iota(jnp.int32, (_SIZE, _SIZE), 0)
    c = lax.broadcasted_iota(jnp.int32, (_SIZE, _SIZE), 1)
    tri = (r < c).astype(jnp.float32)
    new_cs = jnp.dot(slopes, tri, preferred_element_type=jnp.float32)
    centered = new_cs - new_cs[:, _SIZE // 2:_SIZE // 2 + 1]
    a2 = centered + _HALF_GRID
    a2n = jnp.concatenate([a2[:, 1:], a2[:, _SIZE - 1:]], axis=1)
    d = a2n - a2
    ah = lax.bitcast_convert_type(
        a2.astype(jnp.bfloat16).astype(jnp.float32), jnp.int32)
    dh = lax.bitcast_convert_type(
        d.astype(jnp.bfloat16).astype(jnp.float32), jnp.int32)
    tbl_ref[...] = ah | lax.shift_right_logical(dh, 16)


def _sc_body(x_hbm, tbl_hbm, out_hbm, tbl_v, iv,
             spin, spout, sa0, sa1, se0, se1):
    cid = lax.axis_index("c")
    sid = lax.axis_index("s")
    wid = sid * _NC + cid
    pltpu.sync_copy(tbl_hbm, tbl_v)
    sa, se = (sa0, sa1), (se0, se1)

    def xsrc(g):
        return x_hbm.at[pl.ds(g * _CHUNK, _CHUNK)]

    def odst(g):
        return out_hbm.at[pl.ds(g * _CHUNK, _CHUNK)]

    g0w = wid * _CHUNKS_PER_W
    pltpu.async_copy(xsrc(g0w), spin.at[sid, 0], sa0)
    pltpu.async_copy(xsrc(g0w + 1), spin.at[sid, 1], sa1)

    def pair_body(pr, carry):
        for b in (0, 1):
            k = pr * 2 + b
            g = g0w + k
            ch = (g // _CHUNKS_PER_SLAB) % _NUM_ACT
            base_v = jnp.full((16,), ch * _SIZE, jnp.int32)
            sin_b, sout_b = spin.at[sid, b], spout.at[sid, b]
            pltpu.make_async_copy(xsrc(g), sin_b, sa[b]).wait()
            pltpu.sync_copy(sin_b, iv)

            @pl.when(k + 2 < _CHUNKS_PER_W)
            def _():
                pltpu.async_copy(xsrc(g + 2), sin_b, sa[b])

            def vbody(i):
                xv = iv[pl.ds(i, 16)]
                u_raw = xv * _INV_GRID + (_SIZE / 2 - 0.5)
                u_cl = jnp.minimum(jnp.maximum(u_raw, 0.0), float(_SIZE - 3))
                iu = u_cl.astype(jnp.int32)
                fr = u_raw - iu.astype(jnp.float32)
                idx = iu + base_v
                w = plsc.load_gather(tbl_v, [idx])
                a2f = plsc.bitcast(w & jnp.int32(-65536), jnp.float32)
                df = plsc.bitcast(lax.shift_left(w, 16), jnp.float32)
                iv[pl.ds(i, 16)] = a2f + fr * df

            plsc.parallel_loop(0, _CHUNK, step=16, unroll=8)(vbody)

            @pl.when(k >= 2)
            def _():
                pltpu.make_async_copy(sout_b, odst(g - 2), se[b]).wait()

            pltpu.sync_copy(iv, sout_b)
            pltpu.async_copy(sout_b, odst(g), se[b])

        return carry

    lax.fori_loop(0, _CHUNKS_PER_W // 2, pair_body, jnp.int32(0))
    pltpu.make_async_copy(spout.at[sid, 0], odst(g0w + _CHUNKS_PER_W - 2),
                          se0).wait()
    pltpu.make_async_copy(spout.at[sid, 1], odst(g0w + _CHUNKS_PER_W - 1),
                          se1).wait()


def kernel(x, coefficients_vect):
    cs = coefficients_vect.reshape(_NUM_ACT, _SIZE)
    tbl = pl.pallas_call(
        _prep_body,
        out_shape=jax.ShapeDtypeStruct((_NUM_ACT, _SIZE), jnp.int32),
    )(cs)

    mesh = plsc.VectorSubcoreMesh(
        core_axis_name="c", subcore_axis_name="s",
        num_cores=_NC, num_subcores=_NS)
    sc = functools.partial(
        pl.kernel,
        out_type=jax.ShapeDtypeStruct((_N,), jnp.float32),
        mesh=mesh,
        scratch_types=[
            pltpu.VMEM((_TBL,), jnp.int32),
            pltpu.VMEM((_CHUNK,), jnp.float32),
            pltpu.VMEM_SHARED((_NS, 2, _CHUNK), jnp.float32),
            pltpu.VMEM_SHARED((_NS, 2, _CHUNK), jnp.float32),
            pltpu.SemaphoreType.DMA,
            pltpu.SemaphoreType.DMA,
            pltpu.SemaphoreType.DMA,
            pltpu.SemaphoreType.DMA,
        ],
        compiler_params=pltpu.CompilerParams(needs_layout_passes=False),
    )(_sc_body)
    out_flat = sc(x.reshape(_N), tbl.reshape(_TBL))
    return out_flat.reshape(x.shape)


# speedup vs baseline: 1.1855x; 1.1855x over previous
import functools

import jax
import jax.numpy as jnp
from jax import lax
from jax.experimental import pallas as pl
from jax.experimental.pallas import tpu as pltpu
from jax.experimental.pallas import tpu_sc as plsc

_NUM_ACT = 192
_SIZE = 256
_RANGE = 4.0
_GRID = 2.0 * _RANGE / (_SIZE - 1)
_INV_GRID = 1.0 / _GRID
_HALF_GRID = _GRID / 2.0

_H = 224
_SLAB = _H * _H
_N = 2 * _NUM_ACT * _SLAB
_TBL = _NUM_ACT * _SIZE

_NC, _NS = 2, 16
_NW = _NC * _NS
_CHUNK = _SLAB // 4
_CHUNKS_PER_SLAB = _SLAB // _CHUNK
_TOTAL_CHUNKS = _N // _CHUNK
_CHUNKS_PER_W = _TOTAL_CHUNKS // _NW


def _prep_body(cs_ref, tbl_ref):
    cs = cs_ref[...]
    cs_next = jnp.concatenate([cs[:, 1:], cs[:, _SIZE - 1:]], axis=1)
    slopes = jnp.maximum(cs_next - cs, 0.0)
    col = lax.broadcasted_iota(jnp.int32, (_NUM_ACT, _SIZE), 1)
    slopes = jnp.where((col == 0) | (col >= _SIZE - 2), 0.0, slopes)
    r = lax.broadcasted_iota(jnp.int32, (_SIZE, _SIZE), 0)
    c = lax.broadcasted_iota(jnp.int32, (_SIZE, _SIZE), 1)
    tri = (r < c).astype(jnp.float32)
    new_cs = jnp.dot(slopes, tri, preferred_element_type=jnp.float32)
    centered = new_cs - new_cs[:, _SIZE // 2:_SIZE // 2 + 1]
    a2 = centered + _HALF_GRID
    a2n = jnp.concatenate([a2[:, 1:], a2[:, _SIZE - 1:]], axis=1)
    d = a2n - a2
    ah = lax.bitcast_convert_type(
        a2.astype(jnp.bfloat16).astype(jnp.float32), jnp.int32)
    dh = lax.bitcast_convert_type(
        d.astype(jnp.bfloat16).astype(jnp.float32), jnp.int32)
    tbl_ref[...] = ah | lax.shift_right_logical(dh, 16)


def _sc_body(x_hbm, tbl_hbm, out_hbm, tbl_v, in0, in1, ou0, ou1,
             si0, si1, so0, so1):
    wid = lax.axis_index("s") * _NC + lax.axis_index("c")
    pltpu.sync_copy(tbl_hbm, tbl_v)
    ins, ous = (in0, in1), (ou0, ou1)
    sin, sou = (si0, si1), (so0, so1)

    def xsrc(g):
        return x_hbm.at[pl.ds(g * _CHUNK, _CHUNK)]

    def odst(g):
        return out_hbm.at[pl.ds(g * _CHUNK, _CHUNK)]

    g0w = wid * _CHUNKS_PER_W
    pltpu.async_copy(xsrc(g0w), in0, si0)
    pltpu.async_copy(xsrc(g0w + 1), in1, si1)

    def pair_body(p, carry):
        for b in (0, 1):
            k = p * 2 + b
            g = g0w + k
            ch = (g // _CHUNKS_PER_SLAB) % _NUM_ACT
            base_v = jnp.full((16,), ch * _SIZE, jnp.int32)
            iv, ov = ins[b], ous[b]
            pltpu.make_async_copy(xsrc(g), iv, sin[b]).wait()

            @pl.when(k >= 2)
            def _():
                pltpu.make_async_copy(ov, odst(g - 2), sou[b]).wait()

            def vbody(i):
                xv = iv[pl.ds(i, 16)]
                u_raw = xv * _INV_GRID + (_SIZE / 2 - 0.5)
                u_cl = jnp.minimum(jnp.maximum(u_raw, 0.0), float(_SIZE - 3))
                iu = u_cl.astype(jnp.int32)
                fr = u_raw - iu.astype(jnp.float32)
                idx = iu + base_v
                w = plsc.load_gather(tbl_v, [idx])
                a2f = plsc.bitcast(w & jnp.int32(-65536), jnp.float32)
                df = plsc.bitcast(lax.shift_left(w, 16), jnp.float32)
                ov[pl.ds(i, 16)] = a2f + fr * df

            plsc.parallel_loop(0, _CHUNK, step=16, unroll=8)(vbody)
            pltpu.async_copy(ov, odst(g), sou[b])

            @pl.when(k + 2 < _CHUNKS_PER_W)
            def _():
                pltpu.async_copy(xsrc(g + 2), iv, sin[b])

        return carry

    lax.fori_loop(0, _CHUNKS_PER_W // 2, pair_body, jnp.int32(0))
    pltpu.make_async_copy(ou0, odst(g0w + _CHUNKS_PER_W - 2), so0).wait()
    pltpu.make_async_copy(ou1, odst(g0w + _CHUNKS_PER_W - 1), so1).wait()


def kernel(x, coefficients_vect):
    cs = coefficients_vect.reshape(_NUM_ACT, _SIZE)
    tbl = pl.pallas_call(
        _prep_body,
        out_shape=jax.ShapeDtypeStruct((_NUM_ACT, _SIZE), jnp.int32),
    )(cs)

    mesh = plsc.VectorSubcoreMesh(
        core_axis_name="c", subcore_axis_name="s",
        num_cores=_NC, num_subcores=_NS)
    sc = functools.partial(
        pl.kernel,
        out_type=jax.ShapeDtypeStruct((_N,), jnp.float32),
        mesh=mesh,
        scratch_types=[
            pltpu.VMEM((_TBL,), jnp.int32),
            pltpu.VMEM((_CHUNK,), jnp.float32),
            pltpu.VMEM((_CHUNK,), jnp.float32),
            pltpu.VMEM((_CHUNK,), jnp.float32),
            pltpu.VMEM((_CHUNK,), jnp.float32),
            pltpu.SemaphoreType.DMA,
            pltpu.SemaphoreType.DMA,
            pltpu.SemaphoreType.DMA,
            pltpu.SemaphoreType.DMA,
        ],
        compiler_params=pltpu.CompilerParams(needs_layout_passes=False),
    )(_sc_body)
    out_flat = sc(x.reshape(_N), tbl.reshape(_TBL))
    return out_flat.reshape(x.shape)
